# baseline (speedup 1.0000x reference)
import functools

import jax
import jax.numpy as jnp
from jax import lax
from jax.experimental import pallas as pl
from jax.experimental.pallas import tpu as pltpu
from jax.experimental.pallas import tpu_sc as plsc

_NCHUNK = 4
_SBLK = 1024


def _sc_gather(word_table, idx_flat):
    N = idx_flat.shape[0]
    D = word_table.shape[1]
    info = plsc.get_sparse_core_info()
    NW = info.num_cores * info.num_subcores
    per_w = N // NW
    C = 32
    n_chunks = per_w // C

    mesh = plsc.VectorSubcoreMesh(core_axis_name="c", subcore_axis_name="s")

    @functools.partial(
        pl.kernel,
        mesh=mesh,
        out_type=jax.ShapeDtypeStruct((N, D), jnp.float32),
        scratch_types=[
            pltpu.VMEM((per_w,), jnp.int32),
            pltpu.VMEM((C, D), jnp.float32),
            pltpu.VMEM((C, D), jnp.float32),
            pltpu.SemaphoreType.DMA,
            pltpu.SemaphoreType.DMA,
        ],
    )
    def gather_kernel(table_hbm, idx_hbm, out_hbm, idx_v, buf0, buf1, sem0, sem1):
        wid = lax.axis_index("s") * info.num_cores + lax.axis_index("c")
        base = wid * per_w
        bufs = (buf0, buf1)
        sems = (sem0, sem1)

        pltpu.sync_copy(idx_hbm.at[pl.ds(base, per_w)], idx_v)

        def start(i):
            return pltpu.async_copy(
                table_hbm.at[idx_v.at[pl.ds(i * C, C)]], bufs[i % 2], sems[i % 2])

        copies = [None] * n_chunks
        copies[0] = start(0)
        for i in range(n_chunks):
            if i + 1 < n_chunks:
                copies[i + 1] = start(i + 1)
            copies[i].wait()
            pltpu.sync_copy(bufs[i % 2], out_hbm.at[pl.ds(base + i * C, C)])

    return gather_kernel(word_table, idx_flat)


def _tc_add_ln_chunk(prev_out, word_chunk, tt_chunk, pos_table, consts,
                     ci, s_chunk, alias, eps=1e-5):
    B, _, D = word_chunk.shape
    S = s_chunk * _NCHUNK
    s_blocks = s_chunk // _SBLK
    blk0 = ci * s_blocks

    def body(*refs):
        if len(refs) == 6:
            (tt_ref, emb_ref, pos_ref, const_ref, out_ref) = refs[1:]
        else:
            (tt_ref, emb_ref, pos_ref, const_ref, out_ref) = refs
        x = emb_ref[0] + pos_ref[...]
        tt = tt_ref[0]
        t0 = const_ref[2, :][None, :]
        t1 = const_ref[3, :][None, :]
        x = x + jnp.where(tt == 1, t1, t0)
        inv_d = 1.0 / D
        mu = jnp.sum(x, axis=-1, keepdims=True) * inv_d
        ex2 = jnp.sum(x * x, axis=-1, keepdims=True) * inv_d
        var = ex2 - mu * mu
        y = (x - mu) * lax.rsqrt(var + eps)
        out_ref[0] = y * const_ref[0, :][None, :] + const_ref[1, :][None, :]

    in_specs = [
        pl.BlockSpec((1, _SBLK, 1), lambda i, j: (j, i, 0)),
        pl.BlockSpec((1, _SBLK, D), lambda i, j: (j, i, 0)),
        pl.BlockSpec((_SBLK, D), lambda i, j: (blk0 + i, 0)),
        pl.BlockSpec((8, D), lambda i, j: (0, 0)),
    ]
    args = (tt_chunk, word_chunk, pos_table, consts)
    if alias:
        in_specs = [pl.BlockSpec(memory_space=pl.ANY)] + in_specs
        args = (prev_out,) + args
    return pl.pallas_call(
        body,
        grid=(s_blocks, B),
        in_specs=in_specs,
        out_specs=pl.BlockSpec((1, _SBLK, D), lambda i, j: (j, blk0 + i, 0)),
        out_shape=jax.ShapeDtypeStruct((B, S, D), jnp.float32),
        input_output_aliases={0: 0} if alias else {},
    )(*args)


def kernel(input_ids, token_type_ids, word_table, pos_table, type_table,
           ln_gamma, ln_beta):
    B, S = input_ids.shape
    D = word_table.shape[1]
    s_chunk = S // _NCHUNK

    consts = jnp.concatenate(
        [ln_gamma[None, :], ln_beta[None, :], type_table,
         jnp.zeros((4, D), jnp.float32)], axis=0)

    word_chunks = []
    tt_chunks = []
    for ci in range(_NCHUNK):
        s0 = ci * s_chunk
        ids_c = lax.slice(input_ids, (0, s0), (B, s0 + s_chunk))
        word_chunks.append(
            _sc_gather(word_table, ids_c.reshape(B * s_chunk))
            .reshape(B, s_chunk, D))
        tt_c = lax.slice(token_type_ids, (0, s0), (B, s0 + s_chunk))
        tt_chunks.append(tt_c.reshape(B, s_chunk, 1))

    out = None
    for ci in range(_NCHUNK):
        out = _tc_add_ln_chunk(out, word_chunks[ci], tt_chunks[ci],
                               pos_table, consts, ci, s_chunk,
                               alias=ci > 0)
    return out


# speedup vs baseline: 1.0331x; 1.0331x over previous
import functools

import jax
import jax.numpy as jnp
from jax import lax
from jax.experimental import pallas as pl
from jax.experimental.pallas import tpu as pltpu
from jax.experimental.pallas import tpu_sc as plsc

_NCHUNK = 2
_SBLK = 2048


def _sc_gather(word_table, idx_flat):
    N = idx_flat.shape[0]
    D = word_table.shape[1]
    info = plsc.get_sparse_core_info()
    NW = info.num_cores * info.num_subcores
    per_w = N // NW
    C = 32
    n_chunks = per_w // C

    mesh = plsc.VectorSubcoreMesh(core_axis_name="c", subcore_axis_name="s")

    @functools.partial(
        pl.kernel,
        mesh=mesh,
        out_type=jax.ShapeDtypeStruct((N, D), jnp.float32),
        scratch_types=[
            pltpu.VMEM((per_w,), jnp.int32),
            pltpu.VMEM((C, D), jnp.float32),
            pltpu.VMEM((C, D), jnp.float32),
            pltpu.SemaphoreType.DMA,
            pltpu.SemaphoreType.DMA,
        ],
    )
    def gather_kernel(table_hbm, idx_hbm, out_hbm, idx_v, buf0, buf1, sem0, sem1):
        wid = lax.axis_index("s") * info.num_cores + lax.axis_index("c")
        base = wid * per_w
        bufs = (buf0, buf1)
        sems = (sem0, sem1)

        pltpu.sync_copy(idx_hbm.at[pl.ds(base, per_w)], idx_v)

        def start(i):
            return pltpu.async_copy(
                table_hbm.at[idx_v.at[pl.ds(i * C, C)]], bufs[i % 2], sems[i % 2])

        copies = [None] * n_chunks
        copies[0] = start(0)
        for i in range(n_chunks):
            if i + 1 < n_chunks:
                copies[i + 1] = start(i + 1)
            copies[i].wait()
            pltpu.sync_copy(bufs[i % 2], out_hbm.at[pl.ds(base + i * C, C)])

    return gather_kernel(word_table, idx_flat)


def _tc_add_ln_chunk(prev_out, word_chunk, tt_chunk, pos_table, consts,
                     ci, s_chunk, alias, eps=1e-5):
    B, _, D = word_chunk.shape
    S = s_chunk * _NCHUNK
    s_blocks = s_chunk // _SBLK
    blk0 = ci * s_blocks

    def body(*refs):
        if len(refs) == 6:
            (tt_ref, emb_ref, pos_ref, const_ref, out_ref) = refs[1:]
        else:
            (tt_ref, emb_ref, pos_ref, const_ref, out_ref) = refs
        x = emb_ref[0] + pos_ref[...]
        tt = tt_ref[0]
        t0 = const_ref[2, :][None, :]
        t1 = const_ref[3, :][None, :]
        x = x + jnp.where(tt == 1, t1, t0)
        inv_d = 1.0 / D
        mu = jnp.sum(x, axis=-1, keepdims=True) * inv_d
        ex2 = jnp.sum(x * x, axis=-1, keepdims=True) * inv_d
        var = ex2 - mu * mu
        y = (x - mu) * lax.rsqrt(var + eps)
        out_ref[0] = y * const_ref[0, :][None, :] + const_ref[1, :][None, :]

    in_specs = [
        pl.BlockSpec((1, _SBLK, 1), lambda i, j: (j, i, 0)),
        pl.BlockSpec((1, _SBLK, D), lambda i, j: (j, i, 0)),
        pl.BlockSpec((_SBLK, D), lambda i, j: (blk0 + i, 0)),
        pl.BlockSpec((8, D), lambda i, j: (0, 0)),
    ]
    args = (tt_chunk, word_chunk, pos_table, consts)
    if alias:
        in_specs = [pl.BlockSpec(memory_space=pl.ANY)] + in_specs
        args = (prev_out,) + args
    return pl.pallas_call(
        body,
        grid=(s_blocks, B),
        in_specs=in_specs,
        out_specs=pl.BlockSpec((1, _SBLK, D), lambda i, j: (j, blk0 + i, 0)),
        out_shape=jax.ShapeDtypeStruct((B, S, D), jnp.float32),
        input_output_aliases={0: 0} if alias else {},
    )(*args)


def kernel(input_ids, token_type_ids, word_table, pos_table, type_table,
           ln_gamma, ln_beta):
    B, S = input_ids.shape
    D = word_table.shape[1]
    s_chunk = S // _NCHUNK

    consts = jnp.concatenate(
        [ln_gamma[None, :], ln_beta[None, :], type_table,
         jnp.zeros((4, D), jnp.float32)], axis=0)

    word_chunks = []
    tt_chunks = []
    for ci in range(_NCHUNK):
        s0 = ci * s_chunk
        ids_c = lax.slice(input_ids, (0, s0), (B, s0 + s_chunk))
        word_chunks.append(
            _sc_gather(word_table, ids_c.reshape(B * s_chunk))
            .reshape(B, s_chunk, D))
        tt_c = lax.slice(token_type_ids, (0, s0), (B, s0 + s_chunk))
        tt_chunks.append(tt_c.reshape(B, s_chunk, 1))

    out = None
    for ci in range(_NCHUNK):
        out = _tc_add_ln_chunk(out, word_chunks[ci], tt_chunks[ci],
                               pos_table, consts, ci, s_chunk,
                               alias=ci > 0)
    return out
